# baseline (speedup 1.0000x reference)
import functools

import jax
import jax.numpy as jnp
from jax.experimental import pallas as pl
from jax.experimental.pallas import tpu as pltpu

_NEG = -1e30


def _fpgat_kernel(ids_ref, x_ref, dst_ref, wl_ref, wr_ref, wn_ref,
                  out_ref, m_ref, d_ref, c_ref, ra_ref, *, num_steps,
                  block_rows, num_segments, num_chunks):
    k = pl.program_id(0)
    B = num_segments
    R = block_rows
    C = R // num_chunks
    D = x_ref.shape[1]

    @pl.when(k == 0)
    def _init():
        m_ref[...] = jnp.full((1, 1), _NEG, jnp.float32)
        d_ref[...] = jnp.zeros((B, 1), jnp.float32)
        c_ref[...] = jnp.zeros(c_ref.shape, jnp.float32)
        ra_ref[...] = jax.lax.dot_general(
            wr_ref[...], dst_ref[...], (((1,), (1,)), ((), ())),
            preferred_element_type=jnp.float32)

    r_att = ra_ref[...]
    bi = jax.lax.broadcasted_iota(jnp.int16, (B, C), 0)
    ones = jnp.ones((8, C), jnp.bfloat16)

    alphas, mks = [], []
    for j in range(num_chunks):
        x_j = x_ref[pl.ds(j * C, C), :]
        s_j = ids_ref[0, :, pl.ds(j * C, C)]
        left_j = jax.lax.dot_general(
            wl_ref[...], x_j, (((1,), (1,)), ((), ())),
            preferred_element_type=jnp.float32)
        right_j = jax.lax.gather(
            r_att, s_j.astype(jnp.int32)[:, :, None],
            jax.lax.GatherDimensionNumbers(
                offset_dims=(), collapsed_slice_dims=(1,),
                start_index_map=(1,), operand_batching_dims=(0,),
                start_indices_batching_dims=(0,)),
            slice_sizes=(1, 1),
            mode=jax.lax.GatherScatterMode.PROMISE_IN_BOUNDS)
        z_j = left_j + right_j
        a_j = jnp.where(z_j >= 0, z_j, 0.01 * z_j)
        alphas.append(a_j)
        mks.append(jnp.max(a_j, axis=1, keepdims=True))

    mk = mks[0]
    for j in range(1, num_chunks):
        mk = jnp.maximum(mk, mks[j])
    m_old = m_ref[...]
    m_new = jnp.maximum(m_old, mk)
    corr = jnp.exp(m_old - m_new)

    dsum = None
    cupd = None
    for j in range(num_chunks):
        s_j = ids_ref[0, :, pl.ds(j * C, C)]
        e_j = jnp.exp(alphas[j] - m_new)
        eb_j = jnp.broadcast_to(e_j.astype(jnp.bfloat16), (B, C))
        et_j = jnp.where(bi == s_j, eb_j,
                         jnp.bfloat16(0.0))
        xb_j = x_ref[pl.ds(j * C, C), :].astype(jnp.bfloat16)
        ds_j = jax.lax.dot_general(et_j, ones, (((1,), (1,)), ((), ())),
                                   preferred_element_type=jnp.float32)[:, :1]
        cu_j = jax.lax.dot_general(et_j, xb_j, (((1,), (0,)), ((), ())),
                                   preferred_element_type=jnp.float32)
        dsum = ds_j if dsum is None else dsum + ds_j
        cupd = cu_j if cupd is None else cupd + cu_j

    d_ref[...] = d_ref[...] * corr + dsum
    c_ref[...] = c_ref[...] * corr + cupd
    m_ref[...] = m_new

    @pl.when(k == num_steps - 1)
    def _fin():
        dd = jnp.maximum(d_ref[...], 1e-30)
        cn = c_ref[...] / dd
        out_ref[...] = jax.lax.dot_general(
            cn, wn_ref[...], (((1,), (0,)), ((), ())),
            preferred_element_type=jnp.float32)


def kernel(src_feats, dst_feats, segment_ids, w_att_l, w_att_r, W_node):
    N, D = src_feats.shape
    B = dst_feats.shape[0]
    H = W_node.shape[1]

    R = None
    for cand in (25000, 20000, 10000, 4000, 2000, 1000, 500, 250, 200, 125,
                 100, 50, 40, 25, 8, 1):
        if N % cand == 0:
            R = cand
            break
    K = N // R
    num_chunks = max(1, R // 2500) if R % 2500 == 0 else 1

    ids = segment_ids.astype(jnp.int16).reshape(K, 1, R)
    wl = w_att_l.reshape(1, D).astype(jnp.float32)
    wr = w_att_r.reshape(1, D).astype(jnp.float32)

    body = functools.partial(_fpgat_kernel, num_steps=K, block_rows=R,
                             num_segments=B, num_chunks=num_chunks)
    out = pl.pallas_call(
        body,
        grid=(K,),
        in_specs=[
            pl.BlockSpec((1, 1, R), lambda k: (k, 0, 0)),
            pl.BlockSpec((R, D), lambda k: (k, 0)),
            pl.BlockSpec((B, D), lambda k: (0, 0)),
            pl.BlockSpec((1, D), lambda k: (0, 0)),
            pl.BlockSpec((1, D), lambda k: (0, 0)),
            pl.BlockSpec((D, H), lambda k: (0, 0)),
        ],
        out_specs=pl.BlockSpec((B, H), lambda k: (0, 0)),
        out_shape=jax.ShapeDtypeStruct((B, H), jnp.float32),
        scratch_shapes=[
            pltpu.VMEM((1, 1), jnp.float32),
            pltpu.VMEM((B, 1), jnp.float32),
            pltpu.VMEM((B, D), jnp.float32),
            pltpu.VMEM((1, B), jnp.float32),
        ],
    )(ids, src_feats, dst_feats, wl, wr, W_node)
    return out


# speedup vs baseline: 1.0262x; 1.0262x over previous
import functools

import jax
import jax.numpy as jnp
from jax.experimental import pallas as pl
from jax.experimental.pallas import tpu as pltpu

_NEG = -1e30


def _fpgat_kernel(ids_ref, x_ref, dst_ref, wl_ref, wr_ref, wn_ref,
                  out_ref, m_ref, d_ref, c_ref, ra_ref, *, num_steps,
                  block_rows, num_segments, num_chunks):
    k = pl.program_id(0)
    B = num_segments
    R = block_rows
    C = R // num_chunks
    D = x_ref.shape[1]

    @pl.when(k == 0)
    def _init():
        m_ref[...] = jnp.full((1, 1), _NEG, jnp.float32)
        d_ref[...] = jnp.zeros((B, 1), jnp.float32)
        c_ref[...] = jnp.zeros(c_ref.shape, jnp.float32)
        ra_ref[...] = jax.lax.dot_general(
            wr_ref[...], dst_ref[...], (((1,), (1,)), ((), ())),
            preferred_element_type=jnp.float32)

    r_att = ra_ref[...]
    bi = jax.lax.broadcasted_iota(jnp.int16, (B, C), 0)
    ones = jnp.ones((8, C), jnp.bfloat16)

    alphas, mks = [], []
    for j in range(num_chunks):
        x_j = x_ref[pl.ds(j * C, C), :]
        s_j = ids_ref[0, :, pl.ds(j * C, C)]
        left_j = jax.lax.dot_general(
            wl_ref[...], x_j, (((1,), (1,)), ((), ())),
            preferred_element_type=jnp.float32)
        right_j = jax.lax.gather(
            r_att, s_j.astype(jnp.int32)[:, :, None],
            jax.lax.GatherDimensionNumbers(
                offset_dims=(), collapsed_slice_dims=(1,),
                start_index_map=(1,), operand_batching_dims=(0,),
                start_indices_batching_dims=(0,)),
            slice_sizes=(1, 1),
            mode=jax.lax.GatherScatterMode.PROMISE_IN_BOUNDS)
        z_j = left_j + right_j
        a_j = jnp.where(z_j >= 0, z_j, 0.01 * z_j)
        alphas.append(a_j)
        mks.append(jnp.max(a_j, axis=1, keepdims=True))

    mk = mks[0]
    for j in range(1, num_chunks):
        mk = jnp.maximum(mk, mks[j])
    m_old = m_ref[...]
    m_new = jnp.maximum(m_old, mk)
    corr = jnp.exp(m_old - m_new)

    dsum = None
    cupd = None
    for j in range(num_chunks):
        s_j = ids_ref[0, :, pl.ds(j * C, C)]
        e_j = jnp.exp(alphas[j] - m_new)
        eb_j = jnp.broadcast_to(e_j.astype(jnp.bfloat16), (B, C))
        et_j = jnp.where(bi == s_j, eb_j,
                         jnp.bfloat16(0.0))
        xb_j = x_ref[pl.ds(j * C, C), :].astype(jnp.bfloat16)
        ds_j = jax.lax.dot_general(et_j, ones, (((1,), (1,)), ((), ())),
                                   preferred_element_type=jnp.float32)[:, :1]
        cu_j = jax.lax.dot_general(et_j, xb_j, (((1,), (0,)), ((), ())),
                                   preferred_element_type=jnp.float32)
        dsum = ds_j if dsum is None else dsum + ds_j
        cupd = cu_j if cupd is None else cupd + cu_j

    d_ref[...] = d_ref[...] * corr + dsum
    c_ref[...] = c_ref[...] * corr + cupd
    m_ref[...] = m_new

    @pl.when(k == num_steps - 1)
    def _fin():
        dd = jnp.maximum(d_ref[...], 1e-30)
        cn = c_ref[...] / dd
        out_ref[...] = jax.lax.dot_general(
            cn, wn_ref[...], (((1,), (0,)), ((), ())),
            preferred_element_type=jnp.float32)


def kernel(src_feats, dst_feats, segment_ids, w_att_l, w_att_r, W_node):
    N, D = src_feats.shape
    B = dst_feats.shape[0]
    H = W_node.shape[1]

    R = None
    for cand in (25000, 20000, 10000, 4000, 2000, 1000, 500, 250, 200, 125,
                 100, 50, 40, 25, 8, 1):
        if N % cand == 0:
            R = cand
            break
    K = N // R
    num_chunks = max(1, R // 12500) if R % 12500 == 0 else 1

    ids = segment_ids.astype(jnp.int16).reshape(K, 1, R)
    wl = w_att_l.reshape(1, D).astype(jnp.float32)
    wr = w_att_r.reshape(1, D).astype(jnp.float32)

    body = functools.partial(_fpgat_kernel, num_steps=K, block_rows=R,
                             num_segments=B, num_chunks=num_chunks)
    out = pl.pallas_call(
        body,
        grid=(K,),
        in_specs=[
            pl.BlockSpec((1, 1, R), lambda k: (k, 0, 0)),
            pl.BlockSpec((R, D), lambda k: (k, 0)),
            pl.BlockSpec((B, D), lambda k: (0, 0)),
            pl.BlockSpec((1, D), lambda k: (0, 0)),
            pl.BlockSpec((1, D), lambda k: (0, 0)),
            pl.BlockSpec((D, H), lambda k: (0, 0)),
        ],
        out_specs=pl.BlockSpec((B, H), lambda k: (0, 0)),
        out_shape=jax.ShapeDtypeStruct((B, H), jnp.float32),
        scratch_shapes=[
            pltpu.VMEM((1, 1), jnp.float32),
            pltpu.VMEM((B, 1), jnp.float32),
            pltpu.VMEM((B, D), jnp.float32),
            pltpu.VMEM((1, B), jnp.float32),
        ],
    )(ids, src_feats, dst_feats, wl, wr, W_node)
    return out
